# baseline (speedup 1.0000x reference)
import jax
from jax.experimental import pallas as pl
from jax.experimental.pallas import tpu as pltpu


_ROWS = 320000
_CHUNK_ROWS = 5000
_N_CHUNKS = _ROWS // _CHUNK_ROWS
_N_BUF = 8


def _copy_kernel(x_hbm, o_hbm, buf, in_sems, out_sems):
    def in_copy(i, s):
        return pltpu.make_async_copy(
            x_hbm.at[pl.ds(i * _CHUNK_ROWS, _CHUNK_ROWS)], buf.at[s], in_sems.at[s]
        )

    def out_copy(i, s):
        return pltpu.make_async_copy(
            buf.at[s], o_hbm.at[pl.ds(i * _CHUNK_ROWS, _CHUNK_ROWS)], out_sems.at[s]
        )

    for s in range(_N_BUF):
        in_copy(s, s).start()
    for i in range(_N_CHUNKS):
        s = i % _N_BUF
        in_copy(i, s).wait()
        out_copy(i, s).start()
        nxt = i + _N_BUF
        if nxt < _N_CHUNKS:
            out_copy(i, s).wait()
            in_copy(nxt, s).start()
    for i in range(_N_CHUNKS - _N_BUF, _N_CHUNKS):
        out_copy(i, i % _N_BUF).wait()


def kernel(x_j):
    return pl.pallas_call(
        _copy_kernel,
        out_shape=jax.ShapeDtypeStruct(x_j.shape, x_j.dtype),
        in_specs=[pl.BlockSpec(memory_space=pl.ANY)],
        out_specs=pl.BlockSpec(memory_space=pl.ANY),
        scratch_shapes=[
            pltpu.VMEM((_N_BUF, _CHUNK_ROWS, 128), jax.numpy.float32),
            pltpu.SemaphoreType.DMA((_N_BUF,)),
            pltpu.SemaphoreType.DMA((_N_BUF,)),
        ],
    )(x_j)


# speedup vs baseline: 1.0342x; 1.0342x over previous
import jax
from jax.experimental import pallas as pl
from jax.experimental.pallas import tpu as pltpu


_ROWS = 320000
_CHUNK_ROWS = 5000
_N_CHUNKS = _ROWS // _CHUNK_ROWS
_N_BUF = 12
_W = 4


def _copy_kernel(x_hbm, o_hbm, buf, in_sems, out_sems):
    def in_copy(i, s):
        return pltpu.make_async_copy(
            x_hbm.at[pl.ds(i * _CHUNK_ROWS, _CHUNK_ROWS)], buf.at[s], in_sems.at[s]
        )

    def out_copy(i, s):
        return pltpu.make_async_copy(
            buf.at[s], o_hbm.at[pl.ds(i * _CHUNK_ROWS, _CHUNK_ROWS)], out_sems.at[s]
        )

    for s in range(_N_BUF):
        in_copy(s, s).start()
    for i in range(_N_CHUNKS):
        s = i % _N_BUF
        in_copy(i, s).wait()
        out_copy(i, s).start()
        j = i - _W
        if j >= 0 and j + _N_BUF < _N_CHUNKS:
            out_copy(j, j % _N_BUF).wait()
            in_copy(j + _N_BUF, j % _N_BUF).start()
    for i in range(max(_N_CHUNKS - _N_BUF, 0), _N_CHUNKS):
        out_copy(i, i % _N_BUF).wait()


def kernel(x_j):
    return pl.pallas_call(
        _copy_kernel,
        out_shape=jax.ShapeDtypeStruct(x_j.shape, x_j.dtype),
        in_specs=[pl.BlockSpec(memory_space=pl.ANY)],
        out_specs=pl.BlockSpec(memory_space=pl.ANY),
        scratch_shapes=[
            pltpu.VMEM((_N_BUF, _CHUNK_ROWS, 128), jax.numpy.float32),
            pltpu.SemaphoreType.DMA((_N_BUF,)),
            pltpu.SemaphoreType.DMA((_N_BUF,)),
        ],
    )(x_j)


# speedup vs baseline: 1.0463x; 1.0116x over previous
import jax
from jax.experimental import pallas as pl
from jax.experimental.pallas import tpu as pltpu


_ROWS = 320000
_BLOCK_ROWS = 25000


def _copy_kernel(x_ref, o_ref):
    o_ref[...] = x_ref[...]


def kernel(x_j):
    grid = (pl.cdiv(_ROWS, _BLOCK_ROWS),)
    return pl.pallas_call(
        _copy_kernel,
        out_shape=jax.ShapeDtypeStruct(x_j.shape, x_j.dtype),
        grid=grid,
        in_specs=[pl.BlockSpec((_BLOCK_ROWS, 128), lambda i: (i, 0))],
        out_specs=pl.BlockSpec((_BLOCK_ROWS, 128), lambda i: (i, 0)),
    )(x_j)


# speedup vs baseline: 1.0511x; 1.0046x over previous
import jax
from jax.experimental import pallas as pl
from jax.experimental.pallas import tpu as pltpu


_ROWS = 320000
_BLOCK_ROWS = 28000


def _copy_kernel(x_ref, o_ref):
    o_ref[...] = x_ref[...]


def kernel(x_j):
    grid = (pl.cdiv(_ROWS, _BLOCK_ROWS),)
    return pl.pallas_call(
        _copy_kernel,
        out_shape=jax.ShapeDtypeStruct(x_j.shape, x_j.dtype),
        grid=grid,
        in_specs=[pl.BlockSpec((_BLOCK_ROWS, 128), lambda i: (i, 0))],
        out_specs=pl.BlockSpec((_BLOCK_ROWS, 128), lambda i: (i, 0)),
    )(x_j)
